# baseline (speedup 1.0000x reference)
import functools

import jax
import jax.numpy as jnp
from jax import lax
from jax.experimental import pallas as pl
from jax.experimental.pallas import tpu as pltpu
from jax.experimental.pallas import tpu_sc as plsc

N = 8
M = 1024
D = 256
H = 32
COMMITMENT_COST = 0.25

PT = 1024


def _dist_body(x_ref, emb_ref, e2_ref, idx_ref, gidx_ref, loss_ref, hist_ref,
               *, scale):
    n = pl.program_id(0)
    pb = pl.program_id(1)
    bt = x_ref.shape[0]
    x = jnp.swapaxes(x_ref[:, 0], 1, 2).reshape(bt * H, D)
    e = emb_ref[0]
    xb = x.astype(jnp.bfloat16)
    eb = e.astype(jnp.bfloat16)
    s = lax.dot_general(xb, eb, (((1,), (1,)), ((), ())),
                        preferred_element_type=jnp.float32)
    x2 = jnp.sum(x * x, axis=1, keepdims=True)
    e2 = e2_ref[0]
    dist = (e2 + x2) - 2.0 * s
    minval = jnp.min(dist, axis=1, keepdims=True)
    eqm = dist == minval
    iota = lax.broadcasted_iota(jnp.int32, dist.shape, 1)
    idx = jnp.min(jnp.where(eqm, iota, M), axis=1)
    idx_ref[0, 0, 0, :] = idx
    gidx_ref[0, 0, 0, :] = idx + n * M

    cnt = jnp.sum(eqm.astype(jnp.float32), axis=0, keepdims=True)

    @pl.when(pb == 0)
    def _():
        hist_ref[...] = jnp.zeros_like(hist_ref)

    hist_ref[...] += cnt[None]

    @pl.when((n == 0) & (pb == 0))
    def _():
        loss_ref[...] = jnp.zeros_like(loss_ref)

    loss_ref[...] += jnp.sum(minval) * scale


def _transpose_perp_body(q_ref, hist_ref, zq_ref, perp_ref, *, inv_p):
    b = pl.program_id(0)
    for n in range(N):
        zq_ref[0, n] = q_ref[n, 0].T

    @pl.when(b == pl.num_programs(0) - 1)
    def _():
        h = hist_ref[0]
        avg = h * inv_p
        ent = jnp.sum(avg * jnp.log(avg + 1e-10), axis=1, keepdims=True)
        perp_ref[...] = jnp.sum(jnp.exp(-ent)).reshape(1, 1)


def _make_gather(p_total, ch=128):
    nw = 32
    per_w = p_total // nw
    n_ch = per_w // ch
    mesh = plsc.VectorSubcoreMesh(core_axis_name="c", subcore_axis_name="s")

    @functools.partial(
        pl.kernel,
        mesh=mesh,
        out_type=jax.ShapeDtypeStruct((p_total, D), jnp.float32),
        scratch_types=[
            pltpu.VMEM((n_ch, ch), jnp.int32),
            pltpu.VMEM((2, ch, D), jnp.float32),
            pltpu.SemaphoreType.DMA,
            pltpu.SemaphoreType.DMA,
        ],
    )
    def gather_kernel(emb_hbm, gidx_hbm, q_hbm, idx_v, rows_v, gsem, wsem):
        c = lax.axis_index("c")
        s = lax.axis_index("s")
        wid = s * 2 + c
        base = wid * per_w
        pltpu.sync_copy(gidx_hbm.at[wid], idx_v)
        pltpu.async_copy(emb_hbm.at[idx_v.at[0]], rows_v.at[0], gsem).wait()
        for j in range(n_ch):
            bbuf = j % 2
            if j + 1 < n_ch:
                pltpu.make_async_copy(
                    emb_hbm.at[idx_v.at[j + 1]], rows_v.at[1 - bbuf], gsem
                ).start()
            pltpu.make_async_copy(
                rows_v.at[bbuf], q_hbm.at[pl.ds(base + j * ch, ch)], wsem
            ).start()
            if j + 1 < n_ch:
                pltpu.make_async_copy(
                    emb_hbm.at[idx_v.at[j + 1]], rows_v.at[1 - bbuf], gsem
                ).wait()
            pltpu.make_async_copy(
                rows_v.at[bbuf], q_hbm.at[pl.ds(base + j * ch, ch)], wsem
            ).wait()

    return gather_kernel


def kernel(x, embedding):
    bs = x.shape[0]
    p = bs * H
    p_total = N * p
    pb = p // PT

    x4 = x.reshape(bs, N, D, H)
    bt = PT // H

    e2in = jnp.sum(embedding ** 2, axis=2)[:, None, :]
    scale = COMMITMENT_COST / (p_total * D)
    idx4, gidx4, loss2, hist = pl.pallas_call(
        functools.partial(_dist_body, scale=scale),
        grid=(N, pb),
        in_specs=[
            pl.BlockSpec((bt, 1, D, H), lambda n, t: (t, n, 0, 0)),
            pl.BlockSpec((1, M, D), lambda n, t: (n, 0, 0)),
            pl.BlockSpec((1, 1, M), lambda n, t: (n, 0, 0)),
        ],
        out_specs=[
            pl.BlockSpec((1, 1, 1, PT), lambda n, t: (n, t, 0, 0)),
            pl.BlockSpec((1, 1, 1, PT), lambda n, t: (n, t, 0, 0)),
            pl.BlockSpec((1, 1), lambda n, t: (0, 0)),
            pl.BlockSpec((1, 1, M), lambda n, t: (n, 0, 0)),
        ],
        out_shape=[
            jax.ShapeDtypeStruct((N, pb, 1, PT), jnp.int32),
            jax.ShapeDtypeStruct((N, pb, 1, PT), jnp.int32),
            jax.ShapeDtypeStruct((1, 1), jnp.float32),
            jax.ShapeDtypeStruct((N, 1, M), jnp.float32),
        ],
    )(x4, embedding, e2in)

    emb2d = embedding.reshape(N * M, D)
    gidx3 = gidx4.reshape(32, p_total // (32 * 128), 128)
    q_flat = _make_gather(p_total)(emb2d, gidx3)

    q4 = q_flat.reshape(N, bs, H, D)
    hist3 = hist.reshape(1, N, M)
    zq4, perp2 = pl.pallas_call(
        functools.partial(_transpose_perp_body, inv_p=1.0 / p),
        grid=(bs,),
        in_specs=[
            pl.BlockSpec((N, 1, H, D), lambda b: (0, b, 0, 0)),
            pl.BlockSpec((1, N, M), lambda b: (0, 0, 0)),
        ],
        out_specs=[
            pl.BlockSpec((1, N, D, H), lambda b: (b, 0, 0, 0)),
            pl.BlockSpec((1, 1), lambda b: (0, 0)),
        ],
        out_shape=[
            jax.ShapeDtypeStruct((bs, N, D, H), jnp.float32),
            jax.ShapeDtypeStruct((1, 1), jnp.float32),
        ],
    )(q4, hist3)

    z_q = zq4.reshape(bs, N * D * H)
    loss = loss2[0, 0]
    perplexity = perp2[0, 0]
    ind_out = idx4.reshape(N, bs, H, 1).transpose(1, 0, 2, 3)
    return (z_q, loss, perplexity, ind_out)


# speedup vs baseline: 1.3245x; 1.3245x over previous
import functools

import jax
import jax.numpy as jnp
from jax import lax
from jax.experimental import pallas as pl
from jax.experimental.pallas import tpu as pltpu
from jax.experimental.pallas import tpu_sc as plsc

N = 8
M = 1024
D = 256
H = 32
COMMITMENT_COST = 0.25

PT = 1024


def _dist_body(x_ref, emb_ref, e2_ref, idx_ref, gidx_ref, loss_ref, hist_ref,
               *, scale):
    n = pl.program_id(0)
    pb = pl.program_id(1)
    x = x_ref[0]
    e = emb_ref[0]
    xb = x.astype(jnp.bfloat16)
    eb = e.astype(jnp.bfloat16)
    s = lax.dot_general(xb, eb, (((1,), (1,)), ((), ())),
                        preferred_element_type=jnp.float32)
    x2 = jnp.sum(x * x, axis=1, keepdims=True)
    e2 = e2_ref[0]
    dist = (e2 + x2) - 2.0 * s
    minval = jnp.min(dist, axis=1, keepdims=True)
    eqm = dist == minval
    iota = lax.broadcasted_iota(jnp.int32, dist.shape, 1)
    idx = jnp.min(jnp.where(eqm, iota, M), axis=1)
    idx_ref[0, 0, 0, :] = idx
    gidx_ref[0, 0, 0, :] = idx + n * M

    cnt = jnp.sum(eqm.astype(jnp.float32), axis=0, keepdims=True)

    @pl.when(pb == 0)
    def _():
        hist_ref[...] = jnp.zeros_like(hist_ref)

    hist_ref[...] += cnt[None]

    @pl.when((n == 0) & (pb == 0))
    def _():
        loss_ref[...] = jnp.zeros_like(loss_ref)

    loss_ref[...] += jnp.sum(minval) * scale


def _transpose_perp_body(q_ref, hist_ref, zq_ref, perp_ref, *, inv_p):
    b = pl.program_id(0)
    bc = zq_ref.shape[0]
    for bb in range(bc):
        for n in range(N):
            zq_ref[bb, n] = q_ref[n, bb].T

    @pl.when(b == pl.num_programs(0) - 1)
    def _():
        h = hist_ref[0]
        avg = h * inv_p
        ent = jnp.sum(avg * jnp.log(avg + 1e-10), axis=1, keepdims=True)
        perp_ref[...] = jnp.sum(jnp.exp(-ent)).reshape(1, 1)


def _make_gather(p_total, ch=128):
    nw = 32
    per_w = p_total // nw
    n_ch = per_w // ch
    mesh = plsc.VectorSubcoreMesh(core_axis_name="c", subcore_axis_name="s")

    @functools.partial(
        pl.kernel,
        mesh=mesh,
        out_type=jax.ShapeDtypeStruct((p_total, D), jnp.float32),
        scratch_types=[
            pltpu.VMEM((n_ch, ch), jnp.int32),
            pltpu.VMEM((2, ch, D), jnp.float32),
            pltpu.SemaphoreType.DMA,
            pltpu.SemaphoreType.DMA,
        ],
    )
    def gather_kernel(emb_hbm, gidx_hbm, q_hbm, idx_v, rows_v, gsem, wsem):
        c = lax.axis_index("c")
        s = lax.axis_index("s")
        wid = s * 2 + c
        base = wid * per_w
        pltpu.sync_copy(gidx_hbm.at[wid], idx_v)
        pltpu.async_copy(emb_hbm.at[idx_v.at[0]], rows_v.at[0], gsem).wait()
        for j in range(n_ch):
            bbuf = j % 2
            if j + 1 < n_ch:
                pltpu.make_async_copy(
                    emb_hbm.at[idx_v.at[j + 1]], rows_v.at[1 - bbuf], gsem
                ).start()
            pltpu.make_async_copy(
                rows_v.at[bbuf], q_hbm.at[pl.ds(base + j * ch, ch)], wsem
            ).start()
            if j + 1 < n_ch:
                pltpu.make_async_copy(
                    emb_hbm.at[idx_v.at[j + 1]], rows_v.at[1 - bbuf], gsem
                ).wait()
            pltpu.make_async_copy(
                rows_v.at[bbuf], q_hbm.at[pl.ds(base + j * ch, ch)], wsem
            ).wait()

    return gather_kernel


def kernel(x, embedding):
    bs = x.shape[0]
    p = bs * H
    p_total = N * p
    pb = p // PT

    x_flat = x.reshape(bs, N, D, H).transpose(1, 0, 3, 2).reshape(N, p, D)

    e2in = jnp.sum(embedding ** 2, axis=2)[:, None, :]
    scale = COMMITMENT_COST / (p_total * D)
    idx4, gidx4, loss2, hist = pl.pallas_call(
        functools.partial(_dist_body, scale=scale),
        grid=(N, pb),
        in_specs=[
            pl.BlockSpec((1, PT, D), lambda n, t: (n, t, 0)),
            pl.BlockSpec((1, M, D), lambda n, t: (n, 0, 0)),
            pl.BlockSpec((1, 1, M), lambda n, t: (n, 0, 0)),
        ],
        out_specs=[
            pl.BlockSpec((1, 1, 1, PT), lambda n, t: (n, t, 0, 0)),
            pl.BlockSpec((1, 1, 1, PT), lambda n, t: (n, t, 0, 0)),
            pl.BlockSpec((1, 1), lambda n, t: (0, 0)),
            pl.BlockSpec((1, 1, M), lambda n, t: (n, 0, 0)),
        ],
        out_shape=[
            jax.ShapeDtypeStruct((N, pb, 1, PT), jnp.int32),
            jax.ShapeDtypeStruct((N, pb, 1, PT), jnp.int32),
            jax.ShapeDtypeStruct((1, 1), jnp.float32),
            jax.ShapeDtypeStruct((N, 1, M), jnp.float32),
        ],
    )(x_flat, embedding, e2in)

    emb2d = embedding.reshape(N * M, D)
    gidx3 = gidx4.reshape(32, p_total // (32 * 128), 128)
    q_flat = _make_gather(p_total)(emb2d, gidx3)

    q4 = q_flat.reshape(N, bs, H, D)
    hist3 = hist.reshape(1, N, M)
    bc = 8
    zq4, perp2 = pl.pallas_call(
        functools.partial(_transpose_perp_body, inv_p=1.0 / p),
        grid=(bs // bc,),
        in_specs=[
            pl.BlockSpec((N, bc, H, D), lambda b: (0, b, 0, 0)),
            pl.BlockSpec((1, N, M), lambda b: (0, 0, 0)),
        ],
        out_specs=[
            pl.BlockSpec((bc, N, D, H), lambda b: (b, 0, 0, 0)),
            pl.BlockSpec((1, 1), lambda b: (0, 0)),
        ],
        out_shape=[
            jax.ShapeDtypeStruct((bs, N, D, H), jnp.float32),
            jax.ShapeDtypeStruct((1, 1), jnp.float32),
        ],
    )(q4, hist3)

    z_q = zq4.reshape(bs, N * D * H)
    loss = loss2[0, 0]
    perplexity = perp2[0, 0]
    ind_out = idx4.reshape(N, bs, H, 1).transpose(1, 0, 2, 3)
    return (z_q, loss, perplexity, ind_out)


# speedup vs baseline: 1.4122x; 1.0661x over previous
import functools

import jax
import jax.numpy as jnp
from jax import lax
from jax.experimental import pallas as pl
from jax.experimental.pallas import tpu as pltpu
from jax.experimental.pallas import tpu_sc as plsc

N = 8
M = 1024
D = 256
H = 32
COMMITMENT_COST = 0.25

PT = 2048


def _dist_body(x_ref, emb_ref, e2_ref, idx_ref, gidx_ref, loss_ref, hist_ref,
               *, scale):
    n = pl.program_id(0)
    pb = pl.program_id(1)
    x = x_ref[0]
    eb = emb_ref[0]
    xb = x.astype(jnp.bfloat16)
    s = lax.dot_general(xb, eb, (((1,), (1,)), ((), ())),
                        preferred_element_type=jnp.float32)
    x2 = jnp.sum(x * x, axis=1, keepdims=True)
    e2 = e2_ref[0]
    dist = (e2 + x2) - 2.0 * s
    minval = jnp.min(dist, axis=1, keepdims=True)
    eqm = dist == minval
    iota = lax.broadcasted_iota(jnp.int32, dist.shape, 1)
    idx = jnp.min(jnp.where(eqm, iota, M), axis=1)
    idx_ref[0, 0, 0, :] = idx
    gidx_ref[0, 0, 0, :] = idx + n * M

    cnt = jnp.sum(eqm.astype(jnp.float32), axis=0, keepdims=True)

    @pl.when(pb == 0)
    def _():
        hist_ref[...] = jnp.zeros_like(hist_ref)

    hist_ref[...] += cnt[None]

    @pl.when((n == 0) & (pb == 0))
    def _():
        loss_ref[...] = jnp.zeros_like(loss_ref)

    loss_ref[...] += jnp.sum(minval) * scale


def _transpose_perp_body(q_ref, hist_ref, zq_ref, perp_ref, *, inv_p):
    b = pl.program_id(0)
    bc = zq_ref.shape[0]
    for bb in range(bc):
        for n in range(N):
            zq_ref[bb, n] = q_ref[n, bb].T

    @pl.when(b == pl.num_programs(0) - 1)
    def _():
        h = hist_ref[0]
        avg = h * inv_p
        ent = jnp.sum(avg * jnp.log(avg + 1e-10), axis=1, keepdims=True)
        perp_ref[...] = jnp.sum(jnp.exp(-ent)).reshape(1, 1)


def _make_gather(p_total, ch=128):
    nw = 32
    per_w = p_total // nw
    n_ch = per_w // ch
    mesh = plsc.VectorSubcoreMesh(core_axis_name="c", subcore_axis_name="s")

    @functools.partial(
        pl.kernel,
        mesh=mesh,
        out_type=jax.ShapeDtypeStruct((p_total, D), jnp.float32),
        scratch_types=[
            pltpu.VMEM((n_ch, ch), jnp.int32),
            pltpu.VMEM((2, ch, D), jnp.float32),
            pltpu.SemaphoreType.DMA,
            pltpu.SemaphoreType.DMA,
        ],
    )
    def gather_kernel(emb_hbm, gidx_hbm, q_hbm, idx_v, rows_v, gsem, wsem):
        c = lax.axis_index("c")
        s = lax.axis_index("s")
        wid = s * 2 + c
        base = wid * per_w
        pltpu.sync_copy(gidx_hbm.at[wid], idx_v)
        pltpu.async_copy(emb_hbm.at[idx_v.at[0]], rows_v.at[0], gsem).wait()
        for j in range(n_ch):
            bbuf = j % 2
            if j + 1 < n_ch:
                pltpu.make_async_copy(
                    emb_hbm.at[idx_v.at[j + 1]], rows_v.at[1 - bbuf], gsem
                ).start()
            pltpu.make_async_copy(
                rows_v.at[bbuf], q_hbm.at[pl.ds(base + j * ch, ch)], wsem
            ).start()
            if j + 1 < n_ch:
                pltpu.make_async_copy(
                    emb_hbm.at[idx_v.at[j + 1]], rows_v.at[1 - bbuf], gsem
                ).wait()
            pltpu.make_async_copy(
                rows_v.at[bbuf], q_hbm.at[pl.ds(base + j * ch, ch)], wsem
            ).wait()

    return gather_kernel


def kernel(x, embedding):
    bs = x.shape[0]
    p = bs * H
    p_total = N * p
    pb = p // PT

    x_flat = x.reshape(bs, N, D, H).transpose(1, 0, 3, 2).reshape(N, p, D)

    e2in = jnp.sum(embedding ** 2, axis=2)[:, None, :]
    scale = COMMITMENT_COST / (p_total * D)
    idx4, gidx4, loss2, hist = pl.pallas_call(
        functools.partial(_dist_body, scale=scale),
        grid=(N, pb),
        in_specs=[
            pl.BlockSpec((1, PT, D), lambda n, t: (n, t, 0)),
            pl.BlockSpec((1, M, D), lambda n, t: (n, 0, 0)),
            pl.BlockSpec((1, 1, M), lambda n, t: (n, 0, 0)),
        ],
        out_specs=[
            pl.BlockSpec((1, 1, 1, PT), lambda n, t: (n, t, 0, 0)),
            pl.BlockSpec((1, 1, 1, PT), lambda n, t: (n, t, 0, 0)),
            pl.BlockSpec((1, 1), lambda n, t: (0, 0)),
            pl.BlockSpec((1, 1, M), lambda n, t: (n, 0, 0)),
        ],
        out_shape=[
            jax.ShapeDtypeStruct((N, pb, 1, PT), jnp.int32),
            jax.ShapeDtypeStruct((N, pb, 1, PT), jnp.int32),
            jax.ShapeDtypeStruct((1, 1), jnp.float32),
            jax.ShapeDtypeStruct((N, 1, M), jnp.float32),
        ],
    )(x_flat, embedding.astype(jnp.bfloat16), e2in)

    emb2d = embedding.reshape(N * M, D)
    gidx3 = gidx4.reshape(32, p_total // (32 * 128), 128)
    q_flat = _make_gather(p_total)(emb2d, gidx3)

    q4 = q_flat.reshape(N, bs, H, D)
    hist3 = hist.reshape(1, N, M)
    bc = 8
    zq4, perp2 = pl.pallas_call(
        functools.partial(_transpose_perp_body, inv_p=1.0 / p),
        grid=(bs // bc,),
        in_specs=[
            pl.BlockSpec((N, bc, H, D), lambda b: (0, b, 0, 0)),
            pl.BlockSpec((1, N, M), lambda b: (0, 0, 0)),
        ],
        out_specs=[
            pl.BlockSpec((bc, N, D, H), lambda b: (b, 0, 0, 0)),
            pl.BlockSpec((1, 1), lambda b: (0, 0)),
        ],
        out_shape=[
            jax.ShapeDtypeStruct((bs, N, D, H), jnp.float32),
            jax.ShapeDtypeStruct((1, 1), jnp.float32),
        ],
    )(q4, hist3)

    z_q = zq4.reshape(bs, N * D * H)
    loss = loss2[0, 0]
    perplexity = perp2[0, 0]
    ind_out = idx4.reshape(N, bs, H, 1).transpose(1, 0, 2, 3)
    return (z_q, loss, perplexity, ind_out)


# speedup vs baseline: 1.4138x; 1.0012x over previous
import functools

import jax
import jax.numpy as jnp
from jax import lax
from jax.experimental import pallas as pl
from jax.experimental.pallas import tpu as pltpu
from jax.experimental.pallas import tpu_sc as plsc

N = 8
M = 1024
D = 256
H = 32
COMMITMENT_COST = 0.25

PT = 2048


def _dist_body(x_ref, emb_ref, e2_ref, idx_ref, gidx_ref, loss_ref, hist_ref,
               *, scale):
    n = pl.program_id(0)
    pb = pl.program_id(1)
    x = x_ref[0]
    eb = emb_ref[0]
    xb = x.astype(jnp.bfloat16)
    s = lax.dot_general(xb, eb, (((1,), (1,)), ((), ())),
                        preferred_element_type=jnp.float32)
    x2 = jnp.sum(x * x, axis=1, keepdims=True)
    e2 = e2_ref[0]
    dist = (e2 + x2) - 2.0 * s
    minval = jnp.min(dist, axis=1, keepdims=True)
    eqm = dist == minval
    iota = lax.broadcasted_iota(jnp.int32, dist.shape, 1)
    idx = jnp.min(jnp.where(eqm, iota, M), axis=1)
    idx_ref[0, 0, 0, :] = idx
    gidx_ref[0, 0, 0, :] = idx + n * M

    cnt = jnp.sum(eqm.astype(jnp.float32), axis=0, keepdims=True)

    @pl.when(pb == 0)
    def _():
        hist_ref[...] = jnp.zeros_like(hist_ref)

    hist_ref[...] += cnt[None]

    @pl.when((n == 0) & (pb == 0))
    def _():
        loss_ref[...] = jnp.zeros_like(loss_ref)

    loss_ref[...] += jnp.sum(minval) * scale


def _transpose_perp_body(q_ref, hist_ref, zq_ref, perp_ref, *, inv_p):
    b = pl.program_id(0)
    bc = zq_ref.shape[0]
    for bb in range(bc):
        for n in range(N):
            zq_ref[bb, n] = q_ref[n, bb].T

    @pl.when(b == pl.num_programs(0) - 1)
    def _():
        h = hist_ref[0]
        avg = h * inv_p
        ent = jnp.sum(avg * jnp.log(avg + 1e-10), axis=1, keepdims=True)
        perp_ref[...] = jnp.sum(jnp.exp(-ent)).reshape(1, 1)


def _make_gather(p_total, ch=128):
    nw = 32
    per_w = p_total // nw
    n_ch = per_w // ch
    mesh = plsc.VectorSubcoreMesh(core_axis_name="c", subcore_axis_name="s")

    @functools.partial(
        pl.kernel,
        mesh=mesh,
        out_type=jax.ShapeDtypeStruct((p_total, D), jnp.float32),
        scratch_types=[
            pltpu.VMEM((n_ch, ch), jnp.int32),
            pltpu.VMEM((2, ch, D), jnp.float32),
            pltpu.SemaphoreType.DMA,
            pltpu.SemaphoreType.DMA,
        ],
    )
    def gather_kernel(emb_hbm, gidx_hbm, q_hbm, idx_v, rows_v, gsem, wsem):
        c = lax.axis_index("c")
        s = lax.axis_index("s")
        wid = s * 2 + c
        base = wid * per_w
        pltpu.sync_copy(gidx_hbm.at[wid], idx_v)
        pltpu.async_copy(emb_hbm.at[idx_v.at[0]], rows_v.at[0], gsem).wait()
        for j in range(n_ch):
            bbuf = j % 2
            if j + 1 < n_ch:
                pltpu.make_async_copy(
                    emb_hbm.at[idx_v.at[j + 1]], rows_v.at[1 - bbuf], gsem
                ).start()
            pltpu.make_async_copy(
                rows_v.at[bbuf], q_hbm.at[pl.ds(base + j * ch, ch)], wsem
            ).start()
            if j + 1 < n_ch:
                pltpu.make_async_copy(
                    emb_hbm.at[idx_v.at[j + 1]], rows_v.at[1 - bbuf], gsem
                ).wait()
            pltpu.make_async_copy(
                rows_v.at[bbuf], q_hbm.at[pl.ds(base + j * ch, ch)], wsem
            ).wait()

    return gather_kernel


def kernel(x, embedding):
    bs = x.shape[0]
    p = bs * H
    p_total = N * p
    pb = p // PT

    x_flat = x.reshape(bs, N, D, H).transpose(1, 0, 3, 2).reshape(N, p, D)

    e2in = jnp.sum(embedding ** 2, axis=2)[:, None, :]
    scale = COMMITMENT_COST / (p_total * D)
    idx4, gidx4, loss2, hist = pl.pallas_call(
        functools.partial(_dist_body, scale=scale),
        grid=(N, pb),
        in_specs=[
            pl.BlockSpec((1, PT, D), lambda n, t: (n, t, 0)),
            pl.BlockSpec((1, M, D), lambda n, t: (n, 0, 0)),
            pl.BlockSpec((1, 1, M), lambda n, t: (n, 0, 0)),
        ],
        out_specs=[
            pl.BlockSpec((1, 1, 1, PT), lambda n, t: (n, t, 0, 0)),
            pl.BlockSpec((1, 1, 1, PT), lambda n, t: (n, t, 0, 0)),
            pl.BlockSpec((1, 1), lambda n, t: (0, 0)),
            pl.BlockSpec((1, 1, M), lambda n, t: (n, 0, 0)),
        ],
        out_shape=[
            jax.ShapeDtypeStruct((N, pb, 1, PT), jnp.int32),
            jax.ShapeDtypeStruct((N, pb, 1, PT), jnp.int32),
            jax.ShapeDtypeStruct((1, 1), jnp.float32),
            jax.ShapeDtypeStruct((N, 1, M), jnp.float32),
        ],
    )(x_flat, embedding.astype(jnp.bfloat16), e2in)

    emb2d = embedding.reshape(N * M, D)
    gidx3 = gidx4.reshape(32, p_total // (32 * 128), 128)
    q_flat = _make_gather(p_total)(emb2d, gidx3)

    q4 = q_flat.reshape(N, bs, H, D)
    hist3 = hist.reshape(1, N, M)
    bc = 16
    zq4, perp2 = pl.pallas_call(
        functools.partial(_transpose_perp_body, inv_p=1.0 / p),
        grid=(bs // bc,),
        in_specs=[
            pl.BlockSpec((N, bc, H, D), lambda b: (0, b, 0, 0)),
            pl.BlockSpec((1, N, M), lambda b: (0, 0, 0)),
        ],
        out_specs=[
            pl.BlockSpec((bc, N, D, H), lambda b: (b, 0, 0, 0)),
            pl.BlockSpec((1, 1), lambda b: (0, 0)),
        ],
        out_shape=[
            jax.ShapeDtypeStruct((bs, N, D, H), jnp.float32),
            jax.ShapeDtypeStruct((1, 1), jnp.float32),
        ],
    )(q4, hist3)

    z_q = zq4.reshape(bs, N * D * H)
    loss = loss2[0, 0]
    perplexity = perp2[0, 0]
    ind_out = idx4.reshape(N, bs, H, 1).transpose(1, 0, 2, 3)
    return (z_q, loss, perplexity, ind_out)
